# baseline (speedup 1.0000x reference)
import dataclasses
import functools

import jax
import jax.numpy as jnp
from jax import lax
from jax.experimental import pallas as pl
from jax.experimental.pallas import tpu as pltpu
from jax.experimental.pallas import tpu_sc as plsc

_N = 4096
_K = 819
_ROWS = 256
_GRID = _N // _ROWS
_L = 16
_NV = _N // _L
_IMIN_INT = -2147483648


def _imin():
    return jnp.int32(_IMIN_INT)


def _rowsum_body(adj_ref, score_ref):
    score_ref[...] = jnp.sum(adj_ref[...], axis=1)


def _row_sums(adj):
    return pl.pallas_call(
        _rowsum_body,
        grid=(_GRID,),
        in_specs=[pl.BlockSpec((_ROWS, _N), lambda i: (i, 0))],
        out_specs=pl.BlockSpec((_ROWS,), lambda i: (i,)),
        out_shape=jax.ShapeDtypeStruct((_N,), jnp.float32),
    )(adj)


def _topk_mask_sc(scores):
    mesh = plsc.VectorSubcoreMesh(core_axis_name="c", subcore_axis_name="s", num_cores=1)
    cp = pltpu.CompilerParams()
    if "needs_layout_passes" in pltpu.CompilerParams.__dataclass_fields__:
        cp = dataclasses.replace(cp, needs_layout_passes=False)

    @functools.partial(
        pl.kernel,
        mesh=mesh,
        compiler_params=cp,
        out_type=jax.ShapeDtypeStruct((_N,), jnp.float32),
        scratch_types=[
            pltpu.VMEM((_N,), jnp.float32),
            pltpu.VMEM((_N,), jnp.int32),
            pltpu.VMEM((_N,), jnp.float32),
            pltpu.SemaphoreType.DMA,
        ],
    )
    def _mask_kernel(scores_hbm, mask_hbm, sv, kv, mv, sem):
        cid = lax.axis_index("c")
        sid = lax.axis_index("s")

        @pl.when(jnp.logical_and(cid == 0, sid == 0))
        def _():
            pltpu.sync_copy(scores_hbm, sv)

            def key_body(j, carry):
                x = sv[pl.ds(j * _L, _L)]
                bits = plsc.bitcast(x, jnp.int32)
                neg = jnp.bitwise_xor(jnp.bitwise_not(bits), _imin())
                kv[pl.ds(j * _L, _L)] = jnp.where(bits >= 0, bits, neg)
                return carry

            lax.fori_loop(0, _NV, key_body, jnp.int32(0), unroll=8)

            def count_ge(cand_s):
                def body(j, acc):
                    v = kv[pl.ds(j * _L, _L)]
                    return acc + (v >= cand_s).astype(jnp.int32)

                acc = lax.fori_loop(
                    0, _NV, body, jnp.zeros((_L,), jnp.int32), unroll=16
                )
                return jnp.sum(acc)

            def mm_body(j, carry):
                mn, mx = carry
                v = kv[pl.ds(j * _L, _L)]
                return jnp.minimum(mn, v), jnp.maximum(mx, v)

            mn_v, mx_v = lax.fori_loop(
                0,
                _NV,
                mm_body,
                (
                    jnp.full((_L,), 2147483647, jnp.int32),
                    jnp.full((_L,), _IMIN_INT, jnp.int32),
                ),
                unroll=8,
            )
            mn_s = jnp.min(mn_v)
            mx_s = jnp.max(mx_v)
            diff = mn_s ^ mx_s
            fb = lax.bitcast_convert_type(
                jnp.maximum(diff, 1).astype(jnp.float32), jnp.int32
            )
            msb = ((fb >> 23) & 0xFF) - 127
            steps = jnp.where(
                diff == 0, 0, jnp.where(diff < 0, jnp.int32(32), msb + 1)
            )
            low_mask = jnp.where(
                steps >= 32,
                jnp.int32(-1),
                (jnp.int32(1) << jnp.minimum(steps, 31)) - 1,
            )
            prefix0_u = (mx_s ^ _imin()) & ~low_mask

            def bit_step(i, prefix_u):
                cand_u = prefix_u | (jnp.int32(1) << (steps - 1 - i))
                cnt = count_ge(cand_u ^ _imin())
                return lax.select(cnt >= _K, cand_u, prefix_u)

            thresh_u = lax.fori_loop(0, steps, bit_step, prefix0_u)
            thresh_s = thresh_u ^ _imin()

            def gt_body(j, acc):
                v = kv[pl.ds(j * _L, _L)]
                return acc + (v > thresh_s).astype(jnp.int32)

            c1 = jnp.sum(
                lax.fori_loop(
                    0, _NV, gt_body, jnp.zeros((_L,), jnp.int32), unroll=16
                )
            )
            r = jnp.int32(_K) - c1

            def mask_step(j, tie_cnt):
                v = kv[pl.ds(j * _L, _L)]
                gt = v > thresh_s
                eq = v == thresh_s
                eqi = eq.astype(jnp.int32)
                inc = jnp.cumsum(eqi)
                excl = tie_cnt + inc - eqi
                m = jnp.logical_or(gt, jnp.logical_and(eq, excl < r))
                mv[pl.ds(j * _L, _L)] = jnp.where(m, jnp.float32(1.0), jnp.float32(0.0))
                return tie_cnt + jnp.sum(eqi)

            lax.fori_loop(0, _NV, mask_step, jnp.int32(0), unroll=8)
            pltpu.sync_copy(mv, mask_hbm)

    return _mask_kernel(scores)


_BROWS = 512
_BGRID = _N // _BROWS


def _build_body(mrow_ref, mcol_ref, out_ref):
    i = pl.program_id(0)
    mr = mrow_ref[...].reshape(_BROWS, 1)
    mc = mcol_ref[...].reshape(1, _N)
    o = jnp.maximum(mr, mc)
    row = lax.broadcasted_iota(jnp.int32, (_BROWS, _N), 0) + i * _BROWS
    col = lax.broadcasted_iota(jnp.int32, (_BROWS, _N), 1)
    out_ref[...] = jnp.where(row == col, jnp.float32(1.0), o)


def _build(mask):
    return pl.pallas_call(
        _build_body,
        grid=(_BGRID,),
        in_specs=[
            pl.BlockSpec((_BROWS,), lambda i: (i,)),
            pl.BlockSpec((_N,), lambda i: (0,)),
        ],
        out_specs=pl.BlockSpec((_BROWS, _N), lambda i: (i, 0)),
        out_shape=jax.ShapeDtypeStruct((_N, _N), jnp.float32),
    )(mask, mask)


def kernel(H, adj):
    scores = _row_sums(adj)
    mask = _topk_mask_sc(scores)
    return _build(mask)


# speedup vs baseline: 1.0204x; 1.0204x over previous
import dataclasses
import functools

import jax
import jax.numpy as jnp
from jax import lax
from jax.experimental import pallas as pl
from jax.experimental.pallas import tpu as pltpu
from jax.experimental.pallas import tpu_sc as plsc

_N = 4096
_K = 819
_ROWS = 512
_GRID = _N // _ROWS
_L = 16
_NV = _N // _L
_IMIN_INT = -2147483648


def _imin():
    return jnp.int32(_IMIN_INT)


def _rowsum_body(adj_ref, score_ref):
    score_ref[...] = jnp.sum(adj_ref[...], axis=1)


def _row_sums(adj):
    return pl.pallas_call(
        _rowsum_body,
        grid=(_GRID,),
        in_specs=[pl.BlockSpec((_ROWS, _N), lambda i: (i, 0))],
        out_specs=pl.BlockSpec((_ROWS,), lambda i: (i,)),
        out_shape=jax.ShapeDtypeStruct((_N,), jnp.float32),
    )(adj)


def _topk_mask_sc(scores):
    mesh = plsc.VectorSubcoreMesh(core_axis_name="c", subcore_axis_name="s", num_cores=1)
    cp = pltpu.CompilerParams()
    if "needs_layout_passes" in pltpu.CompilerParams.__dataclass_fields__:
        cp = dataclasses.replace(cp, needs_layout_passes=False)

    @functools.partial(
        pl.kernel,
        mesh=mesh,
        compiler_params=cp,
        out_type=jax.ShapeDtypeStruct((_N,), jnp.float32),
        scratch_types=[
            pltpu.VMEM((_N,), jnp.float32),
            pltpu.VMEM((_N,), jnp.int32),
            pltpu.VMEM((_N,), jnp.float32),
            pltpu.SemaphoreType.DMA,
        ],
    )
    def _mask_kernel(scores_hbm, mask_hbm, sv, kv, mv, sem):
        cid = lax.axis_index("c")
        sid = lax.axis_index("s")

        @pl.when(jnp.logical_and(cid == 0, sid == 0))
        def _():
            pltpu.sync_copy(scores_hbm, sv)

            def key_body(j, carry):
                x = sv[pl.ds(j * _L, _L)]
                bits = plsc.bitcast(x, jnp.int32)
                neg = jnp.bitwise_xor(jnp.bitwise_not(bits), _imin())
                kv[pl.ds(j * _L, _L)] = jnp.where(bits >= 0, bits, neg)
                return carry

            lax.fori_loop(0, _NV, key_body, jnp.int32(0), unroll=8)

            def count_ge(cand_s):
                def body(j, acc):
                    v = kv[pl.ds(j * _L, _L)]
                    return acc + (v >= cand_s).astype(jnp.int32)

                acc = lax.fori_loop(
                    0, _NV, body, jnp.zeros((_L,), jnp.int32), unroll=16
                )
                return jnp.sum(acc)

            def mm_body(j, carry):
                mn, mx = carry
                v = kv[pl.ds(j * _L, _L)]
                return jnp.minimum(mn, v), jnp.maximum(mx, v)

            mn_v, mx_v = lax.fori_loop(
                0,
                _NV,
                mm_body,
                (
                    jnp.full((_L,), 2147483647, jnp.int32),
                    jnp.full((_L,), _IMIN_INT, jnp.int32),
                ),
                unroll=8,
            )
            mn_s = jnp.min(mn_v)
            mx_s = jnp.max(mx_v)
            diff = mn_s ^ mx_s
            fb = lax.bitcast_convert_type(
                jnp.maximum(diff, 1).astype(jnp.float32), jnp.int32
            )
            msb = ((fb >> 23) & 0xFF) - 127
            steps = jnp.where(
                diff == 0, 0, jnp.where(diff < 0, jnp.int32(32), msb + 1)
            )
            low_mask = jnp.where(
                steps >= 32,
                jnp.int32(-1),
                (jnp.int32(1) << jnp.minimum(steps, 31)) - 1,
            )
            prefix0_u = (mx_s ^ _imin()) & ~low_mask

            def bit_step(i, prefix_u):
                cand_u = prefix_u | (jnp.int32(1) << (steps - 1 - i))
                cnt = count_ge(cand_u ^ _imin())
                return lax.select(cnt >= _K, cand_u, prefix_u)

            thresh_u = lax.fori_loop(0, steps, bit_step, prefix0_u)
            thresh_s = thresh_u ^ _imin()

            def gt_body(j, acc):
                v = kv[pl.ds(j * _L, _L)]
                return acc + (v > thresh_s).astype(jnp.int32)

            c1 = jnp.sum(
                lax.fori_loop(
                    0, _NV, gt_body, jnp.zeros((_L,), jnp.int32), unroll=16
                )
            )
            r = jnp.int32(_K) - c1

            def mask_step(j, tie_cnt):
                v = kv[pl.ds(j * _L, _L)]
                gt = v > thresh_s
                eq = v == thresh_s
                eqi = eq.astype(jnp.int32)
                inc = jnp.cumsum(eqi)
                excl = tie_cnt + inc - eqi
                m = jnp.logical_or(gt, jnp.logical_and(eq, excl < r))
                mv[pl.ds(j * _L, _L)] = jnp.where(m, jnp.float32(1.0), jnp.float32(0.0))
                return tie_cnt + jnp.sum(eqi)

            lax.fori_loop(0, _NV, mask_step, jnp.int32(0), unroll=8)
            pltpu.sync_copy(mv, mask_hbm)

    return _mask_kernel(scores)


_BROWS = 512
_BGRID = _N // _BROWS


def _build_body(mrow_ref, mcol_ref, out_ref):
    i = pl.program_id(0)
    mr = mrow_ref[...].reshape(_BROWS, 1)
    mc = mcol_ref[...].reshape(1, _N)
    o = jnp.maximum(mr, mc)
    row = lax.broadcasted_iota(jnp.int32, (_BROWS, _N), 0) + i * _BROWS
    col = lax.broadcasted_iota(jnp.int32, (_BROWS, _N), 1)
    out_ref[...] = jnp.where(row == col, jnp.float32(1.0), o)


def _build(mask):
    return pl.pallas_call(
        _build_body,
        grid=(_BGRID,),
        in_specs=[
            pl.BlockSpec((_BROWS,), lambda i: (i,)),
            pl.BlockSpec((_N,), lambda i: (0,)),
        ],
        out_specs=pl.BlockSpec((_BROWS, _N), lambda i: (i, 0)),
        out_shape=jax.ShapeDtypeStruct((_N, _N), jnp.float32),
    )(mask, mask)


def kernel(H, adj):
    scores = _row_sums(adj)
    mask = _topk_mask_sc(scores)
    return _build(mask)
